# Initial kernel scaffold; baseline (speedup 1.0000x reference)
import jax
import jax.numpy as jnp
from jax.experimental import pallas as pl


def kernel(x, f_k, f_c):
    raise NotImplementedError("write your pallas kernel here")


# speedup vs baseline: 12.9778x; 12.9778x over previous
import jax
import jax.numpy as jnp
from jax.experimental import pallas as pl

_BLK = 256
_K = 8
_W = 0.5
_EPS = 1e-8


def _mean_body(x_ref, z_ref):
    z_ref[...] = jnp.mean(x_ref[...], axis=0)


def _rowdot(a, b):
    return jnp.sum(a * b[None, :], axis=1)


def _gram_tiles(z, zb):
    g = jax.lax.dot_general(zb, z, (((1,), (1,)), ((), ())),
                            preferred_element_type=jnp.float32)
    zz = jnp.sum(z * z, axis=1)
    zzb = jnp.sum(zb * zb, axis=1)
    mu = jnp.mean(z, axis=0)
    u = _rowdot(z, mu)
    ub = _rowdot(zb, mu)
    c = jnp.sum(mu * mu)
    k = g - ub[:, None] - u[None, :] + c
    d2 = jnp.maximum(zzb[:, None] + zz[None, :] - 2.0 * g, 0.0)
    return k, d2


def _stats_body(z_ref, zb_ref, t_ref, cnt_ref, rs_ref, rssq_ref):
    k, d2 = _gram_tiles(z_ref[...], zb_ref[...])
    rs_ref[0, 0, :] = jnp.sum(k, axis=1)
    rssq_ref[0, 0, :] = jnp.sum(k * k, axis=1)
    masked = d2
    m = jnp.min(masked, axis=1)
    for _ in range(_K - 1):
        masked = jnp.where(masked <= m[:, None], jnp.inf, masked)
        m = jnp.min(masked, axis=1)
    t_ref[0, 0, :] = m
    cnt_ref[0, 0, :] = jnp.sum((d2 <= m[:, None]).astype(jnp.float32), axis=1)


def _out_body(z_ref, zb_ref, tb_ref, cb_ref, rsb_ref, rs_ref, rssq_ref, w_ref,
              o_ref):
    i = pl.program_id(0)
    n = z_ref.shape[0]
    blk = zb_ref.shape[0]
    k, d2 = _gram_tiles(z_ref[...], zb_ref[...])

    rs_all = rs_ref[0, :]
    fe = jnp.sqrt(jnp.sum(rssq_ref[0, :])) + _EPS
    m_all = rs_all * (1.0 / n)
    m_blk = rsb_ref[0, :] * (1.0 / n)
    mbar = jnp.sum(rs_all) * (1.0 / (n * n))
    k2 = (k - m_blk[:, None] - m_all[None, :] + mbar) / fe
    l1 = jnp.sum(jnp.abs(k2), axis=1) + _EPS
    k3 = k2 / l1[:, None]
    col = jax.lax.broadcasted_iota(jnp.int32, (blk, n), 1)
    row = jax.lax.broadcasted_iota(jnp.int32, (blk, n), 0) + i * blk
    dmask = (col == row).astype(jnp.float32)
    k3 = k3 + 0.01 * dmask
    s = jnp.sum(k3, axis=1) + _EPS
    a2 = k3 / s[:, None]

    a1 = (d2 <= tb_ref[0, :][:, None]).astype(jnp.float32)
    a1 = a1 / (cb_ref[0, :] + _EPS)[:, None]

    w = w_ref[0, :]
    aw = w[0] * a1 + w[1] * a2 + w[2] * dmask
    den = jnp.maximum(jnp.sum(jnp.abs(aw), axis=1), 1e-12)
    o_ref[...] = aw / den[:, None]


def kernel(x, f_k, f_c):
    b, n, d = x.shape
    nb = n // _BLK

    z = pl.pallas_call(
        _mean_body,
        out_shape=jax.ShapeDtypeStruct((n, d), jnp.float32),
    )(x)

    t, cnt, rs, rssq = pl.pallas_call(
        _stats_body,
        grid=(nb,),
        in_specs=[
            pl.BlockSpec((n, d), lambda i: (0, 0)),
            pl.BlockSpec((_BLK, d), lambda i: (i, 0)),
        ],
        out_specs=[pl.BlockSpec((1, 1, _BLK), lambda i: (i, 0, 0))] * 4,
        out_shape=[jax.ShapeDtypeStruct((nb, 1, _BLK), jnp.float32)] * 4,
    )(z, z)
    t = t.reshape(1, n)
    cnt = cnt.reshape(1, n)
    rs = rs.reshape(1, n)
    rssq = rssq.reshape(1, n)

    fk = jnp.asarray(f_k) != 0
    fc = jnp.asarray(f_c) != 0
    both = fk & fc
    w1 = jnp.where(both, _W, jnp.where(fk, 1.0, 0.0))
    w2 = jnp.where(both, 1.0 - _W, jnp.where(fc, 1.0, 0.0))
    wi = jnp.where(fk | fc, 0.0, 1.0)
    w = jnp.stack([w1, w2, wi]).astype(jnp.float32)
    w = jnp.concatenate([w, jnp.zeros((125,), jnp.float32)]).reshape(1, 128)

    out = pl.pallas_call(
        _out_body,
        grid=(nb,),
        in_specs=[
            pl.BlockSpec((n, d), lambda i: (0, 0)),
            pl.BlockSpec((_BLK, d), lambda i: (i, 0)),
            pl.BlockSpec((1, _BLK), lambda i: (0, i)),
            pl.BlockSpec((1, _BLK), lambda i: (0, i)),
            pl.BlockSpec((1, _BLK), lambda i: (0, i)),
            pl.BlockSpec((1, n), lambda i: (0, 0)),
            pl.BlockSpec((1, n), lambda i: (0, 0)),
            pl.BlockSpec((1, 128), lambda i: (0, 0)),
        ],
        out_specs=pl.BlockSpec((_BLK, n), lambda i: (i, 0)),
        out_shape=jax.ShapeDtypeStruct((n, n), jnp.float32),
    )(z, z, t, cnt, rs, rs, rssq, w)
    return out


# speedup vs baseline: 17.4865x; 1.3474x over previous
import jax
import jax.numpy as jnp
from jax.experimental import pallas as pl

_BLK = 256
_K = 8
_W = 0.5
_EPS = 1e-8


def _prep_body(x_ref, z_ref, f_ref):
    z = jnp.mean(x_ref[...], axis=0)
    z_ref[...] = z
    mu = jnp.mean(z, axis=0)
    zc = z - mu[None, :]
    cmat = jax.lax.dot_general(zc, zc, (((0,), (0,)), ((), ())),
                               preferred_element_type=jnp.float32)
    fro = jnp.sqrt(jnp.sum(cmat * cmat))
    f_ref[...] = jnp.reshape(fro, (1, 1))


def _main_body(z_ref, zb_ref, f_ref, w_ref, o_ref):
    i = pl.program_id(0)
    z = z_ref[...]
    zb = zb_ref[...]
    n = z.shape[0]
    blk = zb.shape[0]

    g = jax.lax.dot_general(zb, z, (((1,), (1,)), ((), ())),
                            preferred_element_type=jnp.float32)
    mu = jnp.mean(z, axis=0)
    u = jnp.sum(z * mu[None, :], axis=1)
    ub = jnp.sum(zb * mu[None, :], axis=1)
    c = jnp.sum(mu * mu)
    zz = jnp.sum(z * z, axis=1)
    invfe = 1.0 / (f_ref[0, 0] + _EPS)

    k2 = g + (-ub)[:, None] + (c - u)[None, :]
    l1 = jnp.sum(jnp.abs(k2), axis=1) * invfe + _EPS
    s = (jnp.sum(k2, axis=1) * invfe) / l1 + 0.01 + _EPS

    h = g - 0.5 * zz[None, :]
    m = jnp.max(h, axis=1)
    for _ in range(_K - 1):
        m = jnp.max(jnp.where(h < m[:, None], h, -jnp.inf), axis=1)
    ind = h >= m[:, None]
    cnt = jnp.sum(ind.astype(jnp.float32), axis=1)

    w = w_ref[0, :]
    c1 = w[0] / (cnt + _EPS)
    c2 = (w[1] * invfe) / (l1 * s)
    c3 = w[1] * 0.01 / s + w[2]

    col = jax.lax.broadcasted_iota(jnp.int32, (blk, n), 1)
    row = jax.lax.broadcasted_iota(jnp.int32, (blk, n), 0) + i * blk
    dmask = (col == row).astype(jnp.float32)

    aw = jnp.where(ind, c1[:, None], 0.0) + c2[:, None] * k2 \
        + c3[:, None] * dmask
    den = jnp.maximum(jnp.sum(jnp.abs(aw), axis=1), 1e-12)
    o_ref[...] = aw * (1.0 / den)[:, None]


def kernel(x, f_k, f_c):
    b, n, d = x.shape
    nb = n // _BLK

    z, fro = pl.pallas_call(
        _prep_body,
        out_shape=[
            jax.ShapeDtypeStruct((n, d), jnp.float32),
            jax.ShapeDtypeStruct((1, 1), jnp.float32),
        ],
    )(x)

    fk = jnp.asarray(f_k) != 0
    fc = jnp.asarray(f_c) != 0
    both = fk & fc
    w1 = jnp.where(both, _W, jnp.where(fk, 1.0, 0.0))
    w2 = jnp.where(both, 1.0 - _W, jnp.where(fc, 1.0, 0.0))
    wi = jnp.where(fk | fc, 0.0, 1.0)
    w = jnp.stack([w1, w2, wi]).astype(jnp.float32)
    w = jnp.concatenate([w, jnp.zeros((125,), jnp.float32)]).reshape(1, 128)

    out = pl.pallas_call(
        _main_body,
        grid=(nb,),
        in_specs=[
            pl.BlockSpec((n, d), lambda i: (0, 0)),
            pl.BlockSpec((_BLK, d), lambda i: (i, 0)),
            pl.BlockSpec((1, 1), lambda i: (0, 0)),
            pl.BlockSpec((1, 128), lambda i: (0, 0)),
        ],
        out_specs=pl.BlockSpec((_BLK, n), lambda i: (i, 0)),
        out_shape=jax.ShapeDtypeStruct((n, n), jnp.float32),
    )(z, z, fro, w)
    return out


# speedup vs baseline: 18.4135x; 1.0530x over previous
import jax
import jax.numpy as jnp
from jax.experimental import pallas as pl
from jax.experimental.pallas import tpu as pltpu

_BLK = 256
_K = 8
_W = 0.5
_EPS = 1e-8


def _prep_body(x_ref, z_ref, f_ref):
    z = jnp.mean(x_ref[...], axis=0)
    z_ref[...] = z
    mu = jnp.mean(z, axis=0)
    zc = z - mu[None, :]
    cmat = jax.lax.dot_general(zc, zc, (((0,), (0,)), ((), ())),
                               preferred_element_type=jnp.float32)
    fro = jnp.sqrt(jnp.sum(cmat * cmat))
    f_ref[...] = jnp.reshape(fro, (1, 1))


def _main_body(z_ref, zb_ref, f_ref, w_ref, o_ref):
    i = pl.program_id(0)
    z = z_ref[...]
    zb = zb_ref[...]
    n = z.shape[0]
    blk = zb.shape[0]

    g = jax.lax.dot_general(zb, z, (((1,), (1,)), ((), ())),
                            preferred_element_type=jnp.float32)
    mu = jnp.mean(z, axis=0)
    u = jnp.sum(z * mu[None, :], axis=1)
    ub = jnp.sum(zb * mu[None, :], axis=1)
    c = jnp.sum(mu * mu)
    zz = jnp.sum(z * z, axis=1)
    invfe = 1.0 / (f_ref[0, 0] + _EPS)

    k2 = g + (-ub)[:, None] + (c - u)[None, :]
    l1 = jnp.sum(jnp.abs(k2), axis=1) * invfe + _EPS
    s = (jnp.sum(k2, axis=1) * invfe) / l1 + 0.01 + _EPS

    h = g - 0.5 * zz[None, :]
    m = jnp.max(h, axis=1)
    for _ in range(_K - 1):
        m = jnp.max(jnp.where(h < m[:, None], h, -jnp.inf), axis=1)
    sel = jnp.where(h >= m[:, None], 1.0, 0.0)
    cnt = jnp.sum(sel, axis=1)

    w = w_ref[0, :]
    c1 = w[0] / (cnt + _EPS)
    c2 = (w[1] * invfe) / (l1 * s)
    c3 = w[1] * 0.01 / s + w[2]

    aw = c1[:, None] * sel + c2[:, None] * k2
    zzb = jnp.sum(zb * zb, axis=1)
    awd = c1 + c2 * (zzb - 2.0 * ub + c)
    den = jnp.maximum(
        jnp.sum(jnp.abs(aw), axis=1) - jnp.abs(awd) + jnp.abs(awd + c3),
        1e-12)
    invden = 1.0 / den
    o_ref[...] = aw * invden[:, None]
    col = jax.lax.broadcasted_iota(jnp.int32, (blk, blk), 1)
    row = jax.lax.broadcasted_iota(jnp.int32, (blk, blk), 0)
    eye = (col == row).astype(jnp.float32)
    dcols = pl.ds(i * blk, blk)
    o_ref[:, dcols] = o_ref[:, dcols] + (c3 * invden)[:, None] * eye


def kernel(x, f_k, f_c):
    b, n, d = x.shape
    nb = n // _BLK

    z, fro = pl.pallas_call(
        _prep_body,
        out_shape=[
            jax.ShapeDtypeStruct((n, d), jnp.float32),
            jax.ShapeDtypeStruct((1, 1), jnp.float32),
        ],
    )(x)

    fk = jnp.asarray(f_k) != 0
    fc = jnp.asarray(f_c) != 0
    both = fk & fc
    w1 = jnp.where(both, _W, jnp.where(fk, 1.0, 0.0))
    w2 = jnp.where(both, 1.0 - _W, jnp.where(fc, 1.0, 0.0))
    wi = jnp.where(fk | fc, 0.0, 1.0)
    w = jnp.stack([w1, w2, wi]).astype(jnp.float32)
    w = jnp.concatenate([w, jnp.zeros((125,), jnp.float32)]).reshape(1, 128)

    out = pl.pallas_call(
        _main_body,
        grid=(nb,),
        in_specs=[
            pl.BlockSpec((n, d), lambda i: (0, 0)),
            pl.BlockSpec((_BLK, d), lambda i: (i, 0)),
            pl.BlockSpec((1, 1), lambda i: (0, 0)),
            pl.BlockSpec((1, 128), lambda i: (0, 0)),
        ],
        out_specs=pl.BlockSpec((_BLK, n), lambda i: (i, 0)),
        out_shape=jax.ShapeDtypeStruct((n, n), jnp.float32),
        compiler_params=pltpu.CompilerParams(
            dimension_semantics=("parallel",)),
    )(z, z, fro, w)
    return out


# speedup vs baseline: 19.8012x; 1.0754x over previous
import jax
import jax.numpy as jnp
from jax.experimental import pallas as pl
from jax.experimental.pallas import tpu as pltpu

_BLK = 512
_K = 8
_W = 0.5
_EPS = 1e-8


def _prep_body(x_ref, z_ref, f_ref):
    z = jnp.mean(x_ref[...], axis=0)
    z_ref[...] = z
    mu = jnp.mean(z, axis=0)
    zc = z - mu[None, :]
    cmat = jax.lax.dot_general(zc, zc, (((0,), (0,)), ((), ())),
                               preferred_element_type=jnp.float32)
    fro = jnp.sqrt(jnp.sum(cmat * cmat))
    f_ref[...] = jnp.reshape(fro, (1, 1))


def _main_body(z_ref, zb_ref, f_ref, w_ref, o_ref):
    i = pl.program_id(0)
    z = z_ref[...]
    zb = zb_ref[...]
    n = z.shape[0]
    blk = zb.shape[0]

    g = jax.lax.dot_general(zb, z, (((1,), (1,)), ((), ())),
                            preferred_element_type=jnp.float32)
    mu = jnp.mean(z, axis=0)
    u = jnp.sum(z * mu[None, :], axis=1)
    ub = jnp.sum(zb * mu[None, :], axis=1)
    c = jnp.sum(mu * mu)
    zz = jnp.sum(z * z, axis=1)
    invfe = 1.0 / (f_ref[0, 0] + _EPS)

    k2 = g + (-ub)[:, None] + (c - u)[None, :]
    l1 = jnp.sum(jnp.abs(k2), axis=1) * invfe + _EPS
    s = (jnp.sum(k2, axis=1) * invfe) / l1 + 0.01 + _EPS

    h = g - 0.5 * zz[None, :]
    m = jnp.max(h, axis=1)
    for _ in range(_K - 1):
        m = jnp.max(jnp.where(h < m[:, None], h, -jnp.inf), axis=1)
    sel = jnp.where(h >= m[:, None], 1.0, 0.0)
    cnt = jnp.sum(sel, axis=1)

    w = w_ref[0, :]
    c1 = w[0] / (cnt + _EPS)
    c2 = (w[1] * invfe) / (l1 * s)
    c3 = w[1] * 0.01 / s + w[2]

    aw = c1[:, None] * sel + c2[:, None] * k2
    zzb = jnp.sum(zb * zb, axis=1)
    awd = c1 + c2 * (zzb - 2.0 * ub + c)
    den = jnp.maximum(
        jnp.sum(jnp.abs(aw), axis=1) - jnp.abs(awd) + jnp.abs(awd + c3),
        1e-12)
    invden = 1.0 / den
    o_ref[...] = aw * invden[:, None]
    col = jax.lax.broadcasted_iota(jnp.int32, (blk, blk), 1)
    row = jax.lax.broadcasted_iota(jnp.int32, (blk, blk), 0)
    eye = (col == row).astype(jnp.float32)
    dcols = pl.ds(i * blk, blk)
    o_ref[:, dcols] = o_ref[:, dcols] + (c3 * invden)[:, None] * eye


def kernel(x, f_k, f_c):
    b, n, d = x.shape
    nb = n // _BLK

    z, fro = pl.pallas_call(
        _prep_body,
        out_shape=[
            jax.ShapeDtypeStruct((n, d), jnp.float32),
            jax.ShapeDtypeStruct((1, 1), jnp.float32),
        ],
    )(x)

    fk = jnp.asarray(f_k) != 0
    fc = jnp.asarray(f_c) != 0
    both = fk & fc
    w1 = jnp.where(both, _W, jnp.where(fk, 1.0, 0.0))
    w2 = jnp.where(both, 1.0 - _W, jnp.where(fc, 1.0, 0.0))
    wi = jnp.where(fk | fc, 0.0, 1.0)
    w = jnp.stack([w1, w2, wi]).astype(jnp.float32)
    w = jnp.concatenate([w, jnp.zeros((125,), jnp.float32)]).reshape(1, 128)

    out = pl.pallas_call(
        _main_body,
        grid=(nb,),
        in_specs=[
            pl.BlockSpec((n, d), lambda i: (0, 0)),
            pl.BlockSpec((_BLK, d), lambda i: (i, 0)),
            pl.BlockSpec((1, 1), lambda i: (0, 0)),
            pl.BlockSpec((1, 128), lambda i: (0, 0)),
        ],
        out_specs=pl.BlockSpec((_BLK, n), lambda i: (i, 0)),
        out_shape=jax.ShapeDtypeStruct((n, n), jnp.float32),
        compiler_params=pltpu.CompilerParams(
            dimension_semantics=("parallel",)),
    )(z, z, fro, w)
    return out


# speedup vs baseline: 20.9370x; 1.0574x over previous
import jax
import jax.numpy as jnp
from jax.experimental import pallas as pl
from jax.experimental.pallas import tpu as pltpu

_BLK = 512
_K = 8
_W = 0.5
_EPS = 1e-8


def _prep_body(x_ref, z_ref, f_ref):
    z = jnp.mean(x_ref[...], axis=0)
    z_ref[...] = z
    mu = jnp.mean(z, axis=0)
    zc = z - mu[None, :]
    cmat = jax.lax.dot_general(zc, zc, (((0,), (0,)), ((), ())),
                               preferred_element_type=jnp.float32)
    fro = jnp.sqrt(jnp.sum(cmat * cmat))
    f_ref[...] = jnp.reshape(fro, (1, 1))


def _main_body(z_ref, zb_ref, f_ref, w_ref, o_ref):
    i = pl.program_id(0)
    z = z_ref[...]
    zb = zb_ref[...]
    n = z.shape[0]
    blk = zb.shape[0]

    g = jax.lax.dot_general(zb, z, (((1,), (1,)), ((), ())),
                            preferred_element_type=jnp.float32)
    mu = jnp.mean(z, axis=0)
    u = jnp.sum(z * mu[None, :], axis=1)
    ub = jnp.sum(zb * mu[None, :], axis=1)
    c = jnp.sum(mu * mu)
    zz = jnp.sum(z * z, axis=1)
    invfe = 1.0 / (f_ref[0, 0] + _EPS)

    ones = jnp.ones((n, 1), jnp.float32)

    k2 = g + (-ub)[:, None] + (c - u)[None, :]
    absk2 = jnp.abs(k2)
    l1 = jax.lax.dot_general(absk2, ones, (((1,), (0,)), ((), ())),
                             preferred_element_type=jnp.float32)[:, 0] \
        * invfe + _EPS
    s = 0.01 + _EPS

    h = g - 0.5 * zz[None, :]
    m = jnp.max(h, axis=1)
    for _ in range(_K - 1):
        m = jnp.max(jnp.where(h < m[:, None], h, -jnp.inf), axis=1)
    sel = jnp.where(h >= m[:, None], 1.0, 0.0)
    cnt = jax.lax.dot_general(sel, ones, (((1,), (0,)), ((), ())),
                              preferred_element_type=jnp.float32)[:, 0]

    w = w_ref[0, :]
    c1 = w[0] / (cnt + _EPS)
    c2 = (w[1] * invfe) / (l1 * s)
    c3 = w[1] * 0.01 / s + w[2]

    aw = c1[:, None] * sel + c2[:, None] * k2
    zzb = jnp.sum(zb * zb, axis=1)
    awd = c1 + c2 * (zzb - 2.0 * ub + c)
    absaw = jnp.abs(aw)
    sumaw = jax.lax.dot_general(absaw, ones, (((1,), (0,)), ((), ())),
                                preferred_element_type=jnp.float32)[:, 0]
    den = jnp.maximum(sumaw - jnp.abs(awd) + jnp.abs(awd + c3), 1e-12)
    invden = 1.0 / den
    o_ref[...] = aw * invden[:, None]
    col = jax.lax.broadcasted_iota(jnp.int32, (blk, blk), 1)
    row = jax.lax.broadcasted_iota(jnp.int32, (blk, blk), 0)
    eye = (col == row).astype(jnp.float32)
    dcols = pl.ds(i * blk, blk)
    o_ref[:, dcols] = o_ref[:, dcols] + (c3 * invden)[:, None] * eye


def kernel(x, f_k, f_c):
    b, n, d = x.shape
    nb = n // _BLK

    z, fro = pl.pallas_call(
        _prep_body,
        out_shape=[
            jax.ShapeDtypeStruct((n, d), jnp.float32),
            jax.ShapeDtypeStruct((1, 1), jnp.float32),
        ],
    )(x)

    fk = jnp.asarray(f_k) != 0
    fc = jnp.asarray(f_c) != 0
    both = fk & fc
    w1 = jnp.where(both, _W, jnp.where(fk, 1.0, 0.0))
    w2 = jnp.where(both, 1.0 - _W, jnp.where(fc, 1.0, 0.0))
    wi = jnp.where(fk | fc, 0.0, 1.0)
    w = jnp.stack([w1, w2, wi]).astype(jnp.float32)
    w = jnp.concatenate([w, jnp.zeros((125,), jnp.float32)]).reshape(1, 128)

    out = pl.pallas_call(
        _main_body,
        grid=(nb,),
        in_specs=[
            pl.BlockSpec((n, d), lambda i: (0, 0)),
            pl.BlockSpec((_BLK, d), lambda i: (i, 0)),
            pl.BlockSpec((1, 1), lambda i: (0, 0)),
            pl.BlockSpec((1, 128), lambda i: (0, 0)),
        ],
        out_specs=pl.BlockSpec((_BLK, n), lambda i: (i, 0)),
        out_shape=jax.ShapeDtypeStruct((n, n), jnp.float32),
        compiler_params=pltpu.CompilerParams(
            dimension_semantics=("parallel",)),
    )(z, z, fro, w)
    return out


# speedup vs baseline: 22.0065x; 1.0511x over previous
import jax
import jax.numpy as jnp
from jax.experimental import pallas as pl
from jax.experimental.pallas import tpu as pltpu

_BLK = 512
_K = 8
_W = 0.5
_EPS = 1e-8


def _prep_body(x_ref, z_ref, f_ref):
    z = jnp.mean(x_ref[...], axis=0)
    z_ref[...] = z
    mu = jnp.mean(z, axis=0)
    zc = z - mu[None, :]
    cmat = jax.lax.dot_general(zc, zc, (((0,), (0,)), ((), ())),
                               preferred_element_type=jnp.float32)
    fro = jnp.sqrt(jnp.sum(cmat * cmat))
    f_ref[...] = jnp.reshape(fro, (1, 1))


def _main_body(z_ref, zb_ref, f_ref, w_ref, o_ref):
    i = pl.program_id(0)
    z = z_ref[...]
    zb = zb_ref[...]
    n = z.shape[0]
    blk = zb.shape[0]

    g = jax.lax.dot_general(zb, z, (((1,), (1,)), ((), ())),
                            preferred_element_type=jnp.float32)
    mu = jnp.mean(z, axis=0)
    u = jnp.sum(z * mu[None, :], axis=1)
    ub = jnp.sum(zb * mu[None, :], axis=1)
    c = jnp.sum(mu * mu)
    zz = jnp.sum(z * z, axis=1)
    invfe = 1.0 / (f_ref[0, 0] + _EPS)

    ones = jnp.ones((n, 1), jnp.float32)

    k2 = g + (-ub)[:, None] + (c - u)[None, :]
    absk2 = jnp.abs(k2)
    l1 = jax.lax.dot_general(absk2, ones, (((1,), (0,)), ((), ())),
                             preferred_element_type=jnp.float32)[:, 0] \
        * invfe + _EPS
    s = 0.01 + _EPS

    h = g - 0.5 * zz[None, :]
    m = jnp.max(h, axis=1)
    for _ in range(_K - 1):
        m = jnp.max(jnp.where(h < m[:, None], h, -jnp.inf), axis=1)
    w = w_ref[0, :]
    c1 = w[0] / (float(_K) + _EPS)
    c2 = (w[1] * invfe) / (l1 * s)
    c3 = w[1] * 0.01 / s + w[2]

    c2k2 = c2[:, None] * k2
    aw = jnp.where(h >= m[:, None], c2k2 + c1, c2k2)
    zzb = jnp.sum(zb * zb, axis=1)
    awd = c1 + c2 * (zzb - 2.0 * ub + c)
    absaw = jnp.abs(aw)
    sumaw = jax.lax.dot_general(absaw, ones, (((1,), (0,)), ((), ())),
                                preferred_element_type=jnp.float32)[:, 0]
    den = jnp.maximum(sumaw - jnp.abs(awd) + jnp.abs(awd + c3), 1e-12)
    invden = 1.0 / den
    o_ref[...] = aw * invden[:, None]
    col = jax.lax.broadcasted_iota(jnp.int32, (blk, blk), 1)
    row = jax.lax.broadcasted_iota(jnp.int32, (blk, blk), 0)
    eye = (col == row).astype(jnp.float32)
    dcols = pl.ds(i * blk, blk)
    o_ref[:, dcols] = o_ref[:, dcols] + (c3 * invden)[:, None] * eye


def kernel(x, f_k, f_c):
    b, n, d = x.shape
    nb = n // _BLK

    z, fro = pl.pallas_call(
        _prep_body,
        out_shape=[
            jax.ShapeDtypeStruct((n, d), jnp.float32),
            jax.ShapeDtypeStruct((1, 1), jnp.float32),
        ],
    )(x)

    fk = jnp.asarray(f_k) != 0
    fc = jnp.asarray(f_c) != 0
    both = fk & fc
    w1 = jnp.where(both, _W, jnp.where(fk, 1.0, 0.0))
    w2 = jnp.where(both, 1.0 - _W, jnp.where(fc, 1.0, 0.0))
    wi = jnp.where(fk | fc, 0.0, 1.0)
    w = jnp.stack([w1, w2, wi]).astype(jnp.float32)
    w = jnp.concatenate([w, jnp.zeros((125,), jnp.float32)]).reshape(1, 128)

    out = pl.pallas_call(
        _main_body,
        grid=(nb,),
        in_specs=[
            pl.BlockSpec((n, d), lambda i: (0, 0)),
            pl.BlockSpec((_BLK, d), lambda i: (i, 0)),
            pl.BlockSpec((1, 1), lambda i: (0, 0)),
            pl.BlockSpec((1, 128), lambda i: (0, 0)),
        ],
        out_specs=pl.BlockSpec((_BLK, n), lambda i: (i, 0)),
        out_shape=jax.ShapeDtypeStruct((n, n), jnp.float32),
        compiler_params=pltpu.CompilerParams(
            dimension_semantics=("parallel",)),
    )(z, z, fro, w)
    return out


# speedup vs baseline: 22.1041x; 1.0044x over previous
import jax
import jax.numpy as jnp
from jax.experimental import pallas as pl
from jax.experimental.pallas import tpu as pltpu

_BLK = 512
_K = 8
_W = 0.5
_EPS = 1e-8


def _prep_body(x_ref, fkc_ref, z_ref, s_ref):
    z = jnp.mean(x_ref[...], axis=0)
    z_ref[...] = z
    mu = jnp.mean(z, axis=0)
    zc = z - mu[None, :]
    cmat = jax.lax.dot_general(zc, zc, (((0,), (0,)), ((), ())),
                               preferred_element_type=jnp.float32)
    fro = jnp.sqrt(jnp.sum(cmat * cmat))
    fk = fkc_ref[0, 0] != 0.0
    fc = fkc_ref[0, 1] != 0.0
    both = fk & fc
    w1 = jnp.where(both, _W, jnp.where(fk, 1.0, 0.0))
    w2 = jnp.where(both, 1.0 - _W, jnp.where(fc, 1.0, 0.0))
    wi = jnp.where(fk | fc, 0.0, 1.0)
    lane = jax.lax.broadcasted_iota(jnp.int32, (1, 128), 1)
    out = jnp.where(lane == 0, fro, 0.0)
    out = jnp.where(lane == 1, w1, out)
    out = jnp.where(lane == 2, w2, out)
    out = jnp.where(lane == 3, wi, out)
    s_ref[...] = out


def _main_body(z_ref, zb_ref, s_ref, o_ref):
    i = pl.program_id(0)
    z = z_ref[...]
    zb = zb_ref[...]
    n = z.shape[0]
    blk = zb.shape[0]

    g = jax.lax.dot_general(zb, z, (((1,), (1,)), ((), ())),
                            preferred_element_type=jnp.float32)
    mu = jnp.mean(z, axis=0)
    u = jnp.sum(z * mu[None, :], axis=1)
    ub = jnp.sum(zb * mu[None, :], axis=1)
    c = jnp.sum(mu * mu)
    zz = jnp.sum(z * z, axis=1)
    invfe = 1.0 / (s_ref[0, 0] + _EPS)

    ones = jnp.ones((n, 1), jnp.float32)

    k2 = g + (-ub)[:, None] + (c - u)[None, :]
    absk2 = jnp.abs(k2)
    l1 = jax.lax.dot_general(absk2, ones, (((1,), (0,)), ((), ())),
                             preferred_element_type=jnp.float32)[:, 0] \
        * invfe + _EPS
    s = 0.01 + _EPS

    h = g - 0.5 * zz[None, :]
    m = jnp.max(h, axis=1)
    for _ in range(_K - 1):
        m = jnp.max(jnp.where(h < m[:, None], h, -jnp.inf), axis=1)
    w = s_ref[0, 1:4]
    c1 = w[0] / (float(_K) + _EPS)
    c2 = (w[1] * invfe) / (l1 * s)
    c3 = w[1] * 0.01 / s + w[2]

    c2k2 = c2[:, None] * k2
    aw = jnp.where(h >= m[:, None], c2k2 + c1, c2k2)
    zzb = jnp.sum(zb * zb, axis=1)
    awd = c1 + c2 * (zzb - 2.0 * ub + c)
    absaw = jnp.abs(aw)
    sumaw = jax.lax.dot_general(absaw, ones, (((1,), (0,)), ((), ())),
                                preferred_element_type=jnp.float32)[:, 0]
    den = jnp.maximum(sumaw - jnp.abs(awd) + jnp.abs(awd + c3), 1e-12)
    invden = 1.0 / den
    o_ref[...] = aw * invden[:, None]
    col = jax.lax.broadcasted_iota(jnp.int32, (blk, blk), 1)
    row = jax.lax.broadcasted_iota(jnp.int32, (blk, blk), 0)
    eye = (col == row).astype(jnp.float32)
    dcols = pl.ds(i * blk, blk)
    o_ref[:, dcols] = o_ref[:, dcols] + (c3 * invden)[:, None] * eye


def kernel(x, f_k, f_c):
    b, n, d = x.shape
    nb = n // _BLK

    fkc = jnp.stack([jnp.asarray(f_k), jnp.asarray(f_c)]) \
        .astype(jnp.float32).reshape(1, 2)
    z, sv = pl.pallas_call(
        _prep_body,
        out_shape=[
            jax.ShapeDtypeStruct((n, d), jnp.float32),
            jax.ShapeDtypeStruct((1, 128), jnp.float32),
        ],
    )(x, fkc)

    out = pl.pallas_call(
        _main_body,
        grid=(nb,),
        in_specs=[
            pl.BlockSpec((n, d), lambda i: (0, 0)),
            pl.BlockSpec((_BLK, d), lambda i: (i, 0)),
            pl.BlockSpec((1, 128), lambda i: (0, 0)),
        ],
        out_specs=pl.BlockSpec((_BLK, n), lambda i: (i, 0)),
        out_shape=jax.ShapeDtypeStruct((n, n), jnp.float32),
        compiler_params=pltpu.CompilerParams(
            dimension_semantics=("parallel",)),
    )(z, z, sv)
    return out
